# baseline (speedup 1.0000x reference)
import jax
import jax.numpy as jnp
from jax.experimental import pallas as pl
from jax.experimental.pallas import tpu as pltpu

_B = 1024
_K = 100000
_NS = 4
_BM = 16
_NM = _B // (_BM * _NS)
_QB = _B // _NS // _BM


def _body(s0, s1, s2, s3, v_ref, o0, o1, o2, o3):
    v = v_ref[...]
    for s_ref, o_ref in ((s0, o0), (s1, o1), (s2, o2), (s3, o3)):
        o_ref[...] = jnp.sum(s_ref[...] * v, axis=1, keepdims=True)


_matvec = pl.pallas_call(
    _body,
    grid=(_NM,),
    in_specs=[
        pl.BlockSpec((_BM, _K), lambda b, i=i: (_QB * i + b, 0))
        for i in range(_NS)
    ] + [pl.BlockSpec((1, _K), lambda b: (0, 0))],
    out_specs=[
        pl.BlockSpec((_BM, 1), lambda b: (b, 0)) for _ in range(_NS)
    ],
    out_shape=[
        jax.ShapeDtypeStruct((_B // _NS, 1), jnp.float32)
        for _ in range(_NS)
    ],
    compiler_params=pltpu.CompilerParams(
        dimension_semantics=("arbitrary",)),
)


def kernel(state, values):
    outs = _matvec(state, state, state, state, values.reshape(1, _K))
    return jnp.concatenate(outs, axis=0)
